# baseline (speedup 1.0000x reference)
import functools

import jax
import jax.numpy as jnp
from jax import lax
from jax.experimental import pallas as pl
from jax.experimental.pallas import tpu as pltpu
from jax.experimental.pallas import tpu_sc as plsc

B = 1024
S = 50
C = 26
D = 128
V = 256
LOW = -3.0
HIGH = 3.0

NC = 2
NS = 16
NW = NC * NS
BPW = B // NW

ROW = S * C
NRV = ROW // 16 + 1
BINS = C * V
GB = 4
GROWS = GB * C
NG = BPW // GB


def _tc_quant(x_flat):
    BM = 256 * ROW

    def body(x_ref, o_ref):
        i = pl.program_id(0)
        v = x_ref[...]
        t = ((v - LOW) / (HIGH - LOW)) * float(V - 1)
        q = jnp.clip(jnp.round(t), 0.0, float(V - 1))
        idx = q.astype(jnp.int32)
        p = lax.iota(jnp.int32, BM)
        ch = lax.rem(p, C)
        o_ref[...] = ch * V + idx

    return pl.pallas_call(
        body,
        grid=(B * ROW // BM,),
        in_specs=[pl.BlockSpec((BM,), lambda i: (i,))],
        out_specs=pl.BlockSpec((BM,), lambda i: (i,)),
        out_shape=jax.ShapeDtypeStruct((B * ROW,), jnp.int32),
    )(x_flat)


def _sc_hist(addr_flat):
    mesh = plsc.VectorSubcoreMesh(core_axis_name="c", subcore_axis_name="s")

    @functools.partial(
        pl.kernel,
        out_type=jax.ShapeDtypeStruct((B * C, V), jnp.float32),
        mesh=mesh,
        scratch_types=[
            pltpu.VMEM((BPW * ROW,), jnp.int32),
            pltpu.VMEM((2, GROWS, V), jnp.float32),
            pltpu.SemaphoreType.DMA,
            pltpu.SemaphoreType.DMA,
        ],
        compiler_params=pltpu.CompilerParams(
            needs_layout_passes=False, use_tc_tiling_on_sc=True),
    )
    def hist(a_hbm, cnt_hbm, a_v, bins_v, sem0, sem1):
        wid = lax.axis_index("s") * NC + lax.axis_index("c")
        base_b = wid * BPW
        pltpu.sync_copy(a_hbm.at[pl.ds(base_b * ROW, BPW * ROW)], a_v)

        lane = lax.iota(jnp.int32, 16)
        ones = jnp.full((16,), 1.0, jnp.float32)
        nones = jnp.full((16,), -1.0, jnp.float32)
        zeros = jnp.zeros((16,), jnp.float32)
        sems = (sem0, sem1)
        tail_mask = lane >= 12

        def zero_body(rr, c2):
            for par in range(2):
                for k in range(V // 16):
                    bins_v[par, rr, pl.ds(k * 16, 16)] = zeros
            return c2

        lax.fori_loop(0, GROWS, zero_body, 0)

        def scat_batch(j, par, bb, val):
            aoff = bb * ROW
            for r in range(NRV):
                off = r * 16 if (r + 1) * 16 <= ROW else ROW - 16
                a = a_v[pl.ds(aoff + off, 16)]
                row = lax.shift_right_logical(a, 8) + j * C
                col = lax.bitwise_and(a, V - 1)
                if (r + 1) * 16 <= ROW:
                    plsc.addupdate_scatter(bins_v.at[par], [row, col], val)
                else:
                    plsc.addupdate_scatter(
                        bins_v.at[par], [row, col], val, mask=tail_mask)

        def group_body(i, carry):
            for par in range(2):
                g = i * 2 + par

                @pl.when(i > 0)
                def _clear():
                    pltpu.make_async_copy(
                        bins_v.at[par],
                        cnt_hbm.at[pl.ds(0, GROWS), :],
                        sems[par],
                    ).wait()

                    def sub_body(j, c2):
                        scat_batch(j, par, (g - 2) * GB + j, nones)
                        return c2

                    lax.fori_loop(0, GB, sub_body, 0)

                def add_body(j, c2):
                    scat_batch(j, par, g * GB + j, ones)
                    return c2

                lax.fori_loop(0, GB, add_body, 0)

                pltpu.async_copy(
                    bins_v.at[par],
                    cnt_hbm.at[pl.ds((base_b + g * GB) * C, GROWS), :],
                    sems[par],
                )
            return carry

        lax.fori_loop(0, NG // 2, group_body, 0)

        for par in range(2):
            pltpu.make_async_copy(
                bins_v.at[par],
                cnt_hbm.at[pl.ds(0, GROWS), :],
                sems[par],
            ).wait()

    return hist(addr_flat)


def _tc_matmul(counts2d, weight):
    M = B * C
    BM = 2048

    def body(c_ref, w_ref, o_ref):
        acc = lax.dot_general(
            c_ref[...], w_ref[...],
            dimension_numbers=(((1,), (0,)), ((), ())),
            preferred_element_type=jnp.float32,
            precision=lax.Precision.HIGHEST,
        )
        o_ref[...] = acc / float(S)

    return pl.pallas_call(
        body,
        grid=(M // BM,),
        in_specs=[
            pl.BlockSpec((BM, V), lambda i: (i, 0)),
            pl.BlockSpec((V, D), lambda i: (0, 0)),
        ],
        out_specs=pl.BlockSpec((BM, D), lambda i: (i, 0)),
        out_shape=jax.ShapeDtypeStruct((M, D), jnp.float32),
    )(counts2d, weight)


def kernel(x, weight):
    addr = _tc_quant(x.reshape(-1))
    counts = _sc_hist(addr)
    out2d = _tc_matmul(counts, weight)
    return out2d.reshape(B, C, D)


# speedup vs baseline: 1.2464x; 1.2464x over previous
import functools

import jax
import jax.numpy as jnp
from jax import lax
from jax.experimental import pallas as pl
from jax.experimental.pallas import tpu as pltpu
from jax.experimental.pallas import tpu_sc as plsc

B = 1024
S = 50
C = 26
D = 128
V = 256
LOW = -3.0
HIGH = 3.0

NC = 2
NS = 16
NW = NC * NS
BPW = B // NW

ROW = S * C
ROWP = 1312
NRV = ROWP // 16
BINS = C * V


def _tc_quant(x2):
    BM = 128

    def body(x_ref, o_ref):
        v = x_ref[...]
        t = ((v - LOW) / (HIGH - LOW)) * float(V - 1)
        q = jnp.clip(jnp.round(t), 0.0, float(V - 1))
        idx = q.astype(jnp.int32)
        ch = lax.rem(lax.broadcasted_iota(jnp.int32, (BM, ROW), 1), C)
        o_ref[...] = jnp.zeros((BM, ROWP), jnp.int32)
        o_ref[:, : ROW] = ch * V + idx

    return pl.pallas_call(
        body,
        grid=(B // BM,),
        in_specs=[pl.BlockSpec((BM, ROW), lambda i: (i, 0))],
        out_specs=pl.BlockSpec((BM, ROWP), lambda i: (i, 0)),
        out_shape=jax.ShapeDtypeStruct((B, ROWP), jnp.int32),
    )(x2)


def _sc_hist(addr_flat):
    mesh = plsc.VectorSubcoreMesh(core_axis_name="c", subcore_axis_name="s")
    NBUF = 4

    @functools.partial(
        pl.kernel,
        out_type=jax.ShapeDtypeStruct((B * BINS,), jnp.float32),
        mesh=mesh,
        scratch_types=[
            pltpu.VMEM((BPW * ROWP,), jnp.int32),
            pltpu.VMEM((NBUF * BINS,), jnp.float32),
            pltpu.VMEM_SHARED((BINS,), jnp.float32),
            [pltpu.SemaphoreType.DMA] * NBUF,
            [pltpu.SemaphoreType.DMA] * NBUF,
        ],
        compiler_params=pltpu.CompilerParams(needs_layout_passes=False),
    )
    def hist(a_hbm, cnt_hbm, a_v, bins_v, zsp, osem, zsem):
        sid = lax.axis_index("s")
        wid = sid * NC + lax.axis_index("c")
        base_b = wid * BPW
        pltpu.sync_copy(a_hbm.at[pl.ds(base_b * ROWP, BPW * ROWP)], a_v)

        lane = lax.iota(jnp.int32, 16)
        ones = jnp.full((16,), 1.0, jnp.float32)
        zeros = jnp.zeros((16,), jnp.float32)
        tail_mask = lane < (ROW - (NRV - 1) * 16)

        def zero_body(z, c2):
            for k in range(8):
                bins_v[pl.ds(z * 128 + k * 16, 16)] = zeros
            return c2

        lax.fori_loop(0, NBUF * BINS // 128, zero_body, 0)

        @pl.when(sid == 0)
        def _publish_zeros():
            pltpu.sync_copy(bins_v.at[pl.ds(0, BINS)], zsp)

        plsc.subcore_barrier()

        def scat(bb, pbase):
            aoff = bb * ROWP
            for r in range(NRV):
                a = a_v[pl.ds(aoff + r * 16, 16)] + pbase
                if (r + 1) * 16 <= ROW:
                    plsc.addupdate_scatter(bins_v, [a], ones)
                else:
                    plsc.addupdate_scatter(bins_v, [a], ones, mask=tail_mask)

        def batch_body(i, carry):
            for q in range(NBUF):
                bb = i * NBUF + q
                pbase = q * BINS

                @pl.when(i > 0)
                def _wait_zero():
                    pltpu.make_async_copy(
                        zsp, bins_v.at[pl.ds(pbase, BINS)], zsem[q]).wait()

                scat(bb, pbase)

                pltpu.async_copy(
                    bins_v.at[pl.ds(pbase, BINS)],
                    cnt_hbm.at[pl.ds((base_b + bb) * BINS, BINS)],
                    osem[q],
                )

                r2 = (q + 2) % NBUF
                rbase = r2 * BINS

                @pl.when((i > 0) | (q >= 2))
                def _refill():
                    pltpu.make_async_copy(
                        bins_v.at[pl.ds(rbase, BINS)],
                        cnt_hbm.at[pl.ds(0, BINS)],
                        osem[r2],
                    ).wait()
                    pltpu.async_copy(
                        zsp, bins_v.at[pl.ds(rbase, BINS)], zsem[r2])
            return carry

        lax.fori_loop(0, BPW // NBUF, batch_body, 0)

        for q in range(2):
            pltpu.make_async_copy(
                bins_v.at[pl.ds((q + 2) * BINS, BINS)],
                cnt_hbm.at[pl.ds(0, BINS)],
                osem[q + 2],
            ).wait()
        for q in range(NBUF - 2):
            pltpu.make_async_copy(
                zsp, bins_v.at[pl.ds(q * BINS, BINS)], zsem[q]).wait()

    return hist(addr_flat)


def _tc_matmul(counts2d, weight):
    M = B * C
    BM = 3328

    def body(c_ref, w_ref, o_ref):
        acc = lax.dot_general(
            c_ref[...], w_ref[...],
            dimension_numbers=(((1,), (0,)), ((), ())),
            preferred_element_type=jnp.float32,
            precision=lax.Precision.HIGHEST,
        )
        o_ref[...] = acc / float(S)

    return pl.pallas_call(
        body,
        grid=(M // BM,),
        in_specs=[
            pl.BlockSpec((BM, V), lambda i: (i, 0)),
            pl.BlockSpec((V, D), lambda i: (0, 0)),
        ],
        out_specs=pl.BlockSpec((BM, D), lambda i: (i, 0)),
        out_shape=jax.ShapeDtypeStruct((M, D), jnp.float32),
    )(counts2d, weight)


def kernel(x, weight):
    addr = _tc_quant(x.reshape(B, ROW))
    counts = _sc_hist(addr.reshape(-1))
    out2d = _tc_matmul(counts.reshape(B * C, V), weight)
    return out2d.reshape(B, C, D)
